# baseline (speedup 1.0000x reference)
import functools

import jax
import jax.numpy as jnp
from jax import lax
from jax.experimental import pallas as pl
from jax.experimental.pallas import tpu as pltpu
from jax.experimental.pallas import tpu_sc as plsc

NC = 2
NS = 16
NW = NC * NS
CH = 128


def _sc_mesh():
  return plsc.VectorSubcoreMesh(core_axis_name="c", subcore_axis_name="s")


def _make_hist(n_pad, cw):
  rows = n_pad // NS

  @functools.partial(
      pl.kernel,
      out_type=jax.ShapeDtypeStruct((NC, n_pad), jnp.float32),
      mesh=_sc_mesh(),
      scratch_types=[
          pltpu.VMEM((2, cw // 2, CH), jnp.int32),
          pltpu.VMEM((CH,), jnp.float32),
          pltpu.VMEM((rows,), jnp.float32),
          pltpu.SemaphoreType.DMA,
          pltpu.VMEM_SHARED((n_pad,), jnp.float32),
      ],
  )
  def hist(col_hbm, d_hbm, idx_v, ones_v, zer_v, sem, hist_sh):
    cid = lax.axis_index("c")
    sid = lax.axis_index("s")
    pltpu.sync_copy(col_hbm.at[cid, sid], idx_v)
    for i in range(CH // 16):
      ones_v[pl.ds(i * 16, 16)] = jnp.ones((16,), jnp.float32)

    def zfill(i, carry):
      zer_v[pl.ds(i * 16, 16)] = jnp.zeros((16,), jnp.float32)
      return carry

    lax.fori_loop(0, rows // 16, zfill, 0)
    pltpu.sync_copy(zer_v, hist_sh.at[pl.ds(sid * rows, rows)])
    plsc.subcore_barrier()

    for ph in range(2):
      def body(j, carry):
        pltpu.async_copy(ones_v, hist_sh.at[idx_v.at[ph, j]], sem, add=True)
        return carry

      lax.fori_loop(0, cw // 2, body, 0)

    for ph in range(2):
      def drain(j, carry):
        pltpu.make_async_copy(ones_v, hist_sh.at[idx_v.at[ph, j]], sem).wait()
        return carry

      lax.fori_loop(0, cw // 2, drain, 0)

    plsc.subcore_barrier()
    pltpu.sync_copy(hist_sh.at[pl.ds(sid * rows, rows)],
                    d_hbm.at[cid, pl.ds(sid * rows, rows)])

  return hist


def _make_scatter(n_pad, cw, f):
  rows = n_pad // NS

  @functools.partial(
      pl.kernel,
      out_type=jax.ShapeDtypeStruct((NC, n_pad, f), jnp.float32),
      mesh=_sc_mesh(),
      scratch_types=[
          pltpu.VMEM((cw // 2, CH), jnp.int32),
          pltpu.VMEM((cw // 2, CH), jnp.int32),
          pltpu.SemaphoreType.DMA,
          pltpu.SemaphoreType.DMA,
          pltpu.SemaphoreType.DMA,
          pltpu.SemaphoreType.DMA,
          pltpu.VMEM_SHARED((n_pad, f), jnp.float32),
      ],
  )
  def scatter(xp_hbm, row_hbm, col_hbm, y_hbm,
              rowv, colv, sem0, sem1, sema0, sema1, y_sh):
    cid = lax.axis_index("c")
    sid = lax.axis_index("s")

    def run(buf0, buf1):
      scatter_body(xp_hbm, row_hbm, col_hbm, y_hbm,
                   rowv, colv, buf0, buf1, sem0, sem1, sema0, sema1, y_sh,
                   cid, sid, cw, rows, f)

    pl.run_scoped(
        run,
        pltpu.VMEM((CH, f), jnp.float32),
        pltpu.VMEM((CH, f), jnp.float32),
    )

  return scatter


def scatter_body(xp_hbm, row_hbm, col_hbm, y_hbm,
                 rowv, colv, buf0, buf1, sem0, sem1, sema0, sema1, y_sh,
                 cid, sid, cw, rows, f):
    zr = 64

    def zfill(r, carry):
      for i in range(f // 16):
        buf0[r, pl.ds(i * 16, 16)] = jnp.zeros((16,), jnp.float32)
      return carry

    lax.fori_loop(0, zr, zfill, 0)

    def zcopy(r, carry):
      pltpu.sync_copy(buf0.at[pl.ds(0, zr)],
                      y_sh.at[pl.ds(sid * rows + r * zr, zr)])
      return carry

    lax.fori_loop(0, rows // zr, zcopy, 0)
    plsc.subcore_barrier()

    npair = cw // 4

    for ph in range(2):
      pltpu.sync_copy(row_hbm.at[cid, sid, ph], rowv)
      pltpu.sync_copy(col_hbm.at[cid, sid, ph], colv)
      pltpu.async_copy(xp_hbm.at[rowv.at[0]], buf0, sem0)
      pltpu.async_copy(xp_hbm.at[rowv.at[1]], buf1, sem1)

      def body(g, carry):
        j0 = 2 * g

        pltpu.make_async_copy(xp_hbm.at[rowv.at[j0]], buf0, sem0).wait()
        pltpu.async_copy(buf0, y_sh.at[colv.at[j0]], sema0, add=True)

        pltpu.make_async_copy(xp_hbm.at[rowv.at[j0 + 1]], buf1, sem1).wait()
        pltpu.async_copy(buf1, y_sh.at[colv.at[j0 + 1]], sema1, add=True)

        pltpu.make_async_copy(buf0, y_sh.at[colv.at[j0]], sema0).wait()

        @pl.when(g + 1 < npair)
        def _():
          pltpu.async_copy(xp_hbm.at[rowv.at[j0 + 2]], buf0, sem0)

        pltpu.make_async_copy(buf1, y_sh.at[colv.at[j0 + 1]], sema1).wait()

        @pl.when(g + 1 < npair)
        def _():
          pltpu.async_copy(xp_hbm.at[rowv.at[j0 + 3]], buf1, sem1)

        return carry

      lax.fori_loop(0, npair, body, 0)

    plsc.subcore_barrier()
    pltpu.sync_copy(y_sh.at[pl.ds(sid * rows, rows)],
                    y_hbm.at[cid, pl.ds(sid * rows, rows)])


def _prep_body(d_ref, x_ref, rs_ref, xp_ref):
  d = d_ref[0, :] + d_ref[1, :]
  rs = jnp.where(d > 0.0, lax.rsqrt(d), 0.0)
  rs2 = rs[:, None]
  rs_ref[...] = rs2
  xp_ref[...] = rs2 * x_ref[...]


def _final_body(x_ref, yp_ref, rs_ref, e_ref, w_ref, o_ref, *, f, k):
  x = x_ref[...]
  y = rs_ref[...] * (yp_ref[0] + yp_ref[1])
  w = w_ref[...]
  e = e_ref[...]
  acc = x
  for i in range(k):
    hk = jnp.dot(y, w[i, :f, :], preferred_element_type=jnp.float32)
    hk = hk + jnp.dot(x, w[i, f:, :], preferred_element_type=jnp.float32)
    acc = acc + e[:, i:i + 1] * hk
  o_ref[...] = acc


@jax.jit
def kernel(x, adj, e, W):
  n, f = x.shape
  k = W.shape[0]
  eN = adj.shape[1]

  n_pad = ((n + 1024) // 1024) * 1024
  blk = 1024
  e_chunk = NW * CH
  cw = (eN + e_chunk - 1) // e_chunk
  cw = ((cw + 3) // 4) * 4
  e_pad = cw * e_chunk

  pad_idx = (n + jnp.arange(e_pad - eN, dtype=jnp.int32) % (n_pad - n)).astype(jnp.int32)
  row_flat = jnp.concatenate([adj[0], pad_idx])
  col_flat = jnp.concatenate([adj[1], pad_idx])
  row = row_flat.reshape(NC, NS, 2, cw // 2, CH)
  col = col_flat.reshape(NC, NS, 2, cw // 2, CH)
  x_pad = jnp.pad(x, ((0, n_pad - n), (0, 0)))
  e_pad_arr = jnp.pad(e, ((0, n_pad - n), (0, 0)))

  d_part = _make_hist(n_pad, cw)(col)

  rs, xp = pl.pallas_call(
      _prep_body,
      grid=(n_pad // blk,),
      in_specs=[
          pl.BlockSpec((NC, blk), lambda i: (0, i)),
          pl.BlockSpec((blk, f), lambda i: (i, 0)),
      ],
      out_specs=[
          pl.BlockSpec((blk, 1), lambda i: (i, 0)),
          pl.BlockSpec((blk, f), lambda i: (i, 0)),
      ],
      out_shape=[
          jax.ShapeDtypeStruct((n_pad, 1), jnp.float32),
          jax.ShapeDtypeStruct((n_pad, f), jnp.float32),
      ],
  )(d_part, x_pad)

  y_part = _make_scatter(n_pad, cw, f)(xp, row, col)

  out = pl.pallas_call(
      functools.partial(_final_body, f=f, k=k),
      grid=(n_pad // blk,),
      in_specs=[
          pl.BlockSpec((blk, f), lambda i: (i, 0)),
          pl.BlockSpec((NC, blk, f), lambda i: (0, i, 0)),
          pl.BlockSpec((blk, 1), lambda i: (i, 0)),
          pl.BlockSpec((blk, k), lambda i: (i, 0)),
          pl.BlockSpec((k, 2 * f, f), lambda i: (0, 0, 0)),
      ],
      out_specs=pl.BlockSpec((blk, f), lambda i: (i, 0)),
      out_shape=jax.ShapeDtypeStruct((n, f), jnp.float32),
  )(x_pad, y_part, rs, e_pad_arr, W)

  return out


# speedup vs baseline: 1.2012x; 1.2012x over previous
import functools

import jax
import jax.numpy as jnp
from jax import lax
from jax.experimental import pallas as pl
from jax.experimental.pallas import tpu as pltpu
from jax.experimental.pallas import tpu_sc as plsc

NC = 2
NS = 16
NW = NC * NS
CH = 128


def _sc_mesh():
  return plsc.VectorSubcoreMesh(core_axis_name="c", subcore_axis_name="s")


def _make_hist(n_pad, cw):
  rows = n_pad // NS

  @functools.partial(
      pl.kernel,
      out_type=jax.ShapeDtypeStruct((NC, n_pad), jnp.float32),
      mesh=_sc_mesh(),
      scratch_types=[
          pltpu.VMEM((2, cw // 2, CH), jnp.int32),
          pltpu.VMEM((CH,), jnp.float32),
          pltpu.VMEM((rows,), jnp.float32),
          pltpu.SemaphoreType.DMA,
          pltpu.VMEM_SHARED((n_pad,), jnp.float32),
      ],
  )
  def hist(col_hbm, d_hbm, idx_v, ones_v, zer_v, sem, hist_sh):
    cid = lax.axis_index("c")
    sid = lax.axis_index("s")
    pltpu.sync_copy(col_hbm.at[cid, sid], idx_v)
    for i in range(CH // 16):
      ones_v[pl.ds(i * 16, 16)] = jnp.ones((16,), jnp.float32)

    def zfill(i, carry):
      zer_v[pl.ds(i * 16, 16)] = jnp.zeros((16,), jnp.float32)
      return carry

    lax.fori_loop(0, rows // 16, zfill, 0)
    pltpu.sync_copy(zer_v, hist_sh.at[pl.ds(sid * rows, rows)])
    plsc.subcore_barrier()

    for ph in range(2):
      def body(j, carry):
        pltpu.async_copy(ones_v, hist_sh.at[idx_v.at[ph, j]], sem, add=True)
        return carry

      lax.fori_loop(0, cw // 2, body, 0)

    for ph in range(2):
      def drain(j, carry):
        pltpu.make_async_copy(ones_v, hist_sh.at[idx_v.at[ph, j]], sem).wait()
        return carry

      lax.fori_loop(0, cw // 2, drain, 0)

    plsc.subcore_barrier()
    pltpu.sync_copy(hist_sh.at[pl.ds(sid * rows, rows)],
                    d_hbm.at[cid, pl.ds(sid * rows, rows)])

  return hist


def _make_scatter(n_pad, cw, f):
  rows = n_pad // NS

  @functools.partial(
      pl.kernel,
      out_type=jax.ShapeDtypeStruct((NC, n_pad, f), jnp.float32),
      mesh=_sc_mesh(),
      scratch_types=[
          pltpu.VMEM((cw // 2, CH), jnp.int32),
          pltpu.VMEM((cw // 2, CH), jnp.int32),
          pltpu.SemaphoreType.DMA,
          pltpu.SemaphoreType.DMA,
          pltpu.VMEM_SHARED((n_pad, f), jnp.float32),
      ],
  )
  def scatter(xp_hbm, row_hbm, col_hbm, y_hbm,
              rowv, colv, sem0, sem1, y_sh):
    cid = lax.axis_index("c")
    sid = lax.axis_index("s")

    def run(buf0, buf1):
      scatter_body(xp_hbm, row_hbm, col_hbm, y_hbm,
                   rowv, colv, buf0, buf1, sem0, sem1, y_sh,
                   cid, sid, cw, rows, f)

    pl.run_scoped(
        run,
        pltpu.VMEM((CH, f), jnp.float32),
        pltpu.VMEM((CH, f), jnp.float32),
    )

  return scatter


def scatter_body(xp_hbm, row_hbm, col_hbm, y_hbm,
                 rowv, colv, buf0, buf1, sem0, sem1, y_sh,
                 cid, sid, cw, rows, f):
    zr = 64

    def zfill(r, carry):
      for i in range(f // 16):
        buf0[r, pl.ds(i * 16, 16)] = jnp.zeros((16,), jnp.float32)
      return carry

    lax.fori_loop(0, zr, zfill, 0)

    def zcopy(r, carry):
      pltpu.sync_copy(buf0.at[pl.ds(0, zr)],
                      y_sh.at[pl.ds(sid * rows + r * zr, zr)])
      return carry

    lax.fori_loop(0, rows // zr, zcopy, 0)
    plsc.subcore_barrier()

    npair = cw // 4

    for ph in range(2):
      pltpu.sync_copy(row_hbm.at[cid, sid, ph], rowv)
      pltpu.sync_copy(col_hbm.at[cid, sid, ph], colv)
      pltpu.async_copy(xp_hbm.at[rowv.at[0]], buf0, sem0)
      pltpu.async_copy(xp_hbm.at[rowv.at[1]], buf1, sem1)

      def body(g, carry):
        j0 = 2 * g

        pltpu.make_async_copy(xp_hbm.at[rowv.at[j0]], buf0, sem0).wait()
        pltpu.sync_copy(buf0, y_sh.at[colv.at[j0]], add=True)

        @pl.when(g + 1 < npair)
        def _():
          pltpu.async_copy(xp_hbm.at[rowv.at[j0 + 2]], buf0, sem0)

        pltpu.make_async_copy(xp_hbm.at[rowv.at[j0 + 1]], buf1, sem1).wait()
        pltpu.sync_copy(buf1, y_sh.at[colv.at[j0 + 1]], add=True)

        @pl.when(g + 1 < npair)
        def _():
          pltpu.async_copy(xp_hbm.at[rowv.at[j0 + 3]], buf1, sem1)

        return carry

      lax.fori_loop(0, npair, body, 0)

    plsc.subcore_barrier()
    pltpu.sync_copy(y_sh.at[pl.ds(sid * rows, rows)],
                    y_hbm.at[cid, pl.ds(sid * rows, rows)])


def _prep_body(d_ref, x_ref, rs_ref, xp_ref):
  d = d_ref[0, :] + d_ref[1, :]
  rs = jnp.where(d > 0.0, lax.rsqrt(d), 0.0)
  rs2 = rs[:, None]
  rs_ref[...] = rs2
  xp_ref[...] = rs2 * x_ref[...]


def _xw_body(x_ref, e_ref, w_ref, o_ref, *, f, k):
  x = x_ref[...]
  w = w_ref[...]
  e = e_ref[...]
  acc = x
  for i in range(k):
    hk = jnp.dot(x, w[i, f:, :], preferred_element_type=jnp.float32)
    acc = acc + e[:, i:i + 1] * hk
  o_ref[...] = acc


def _final_body(xw_ref, yp_ref, rs_ref, e_ref, w_ref, o_ref, *, f, k):
  y = rs_ref[...] * (yp_ref[0] + yp_ref[1])
  w = w_ref[...]
  e = e_ref[...]
  acc = xw_ref[...]
  for i in range(k):
    hk = jnp.dot(y, w[i, :f, :], preferred_element_type=jnp.float32)
    acc = acc + e[:, i:i + 1] * hk
  o_ref[...] = acc


@jax.jit
def kernel(x, adj, e, W):
  n, f = x.shape
  k = W.shape[0]
  eN = adj.shape[1]

  n_pad = ((n + 1024) // 1024) * 1024
  blk = 1024
  e_chunk = NW * CH
  cw = (eN + e_chunk - 1) // e_chunk
  cw = ((cw + 3) // 4) * 4
  e_pad = cw * e_chunk

  pad_idx = (n + jnp.arange(e_pad - eN, dtype=jnp.int32) % (n_pad - n)).astype(jnp.int32)
  row_flat = jnp.concatenate([adj[0], pad_idx])
  col_flat = jnp.concatenate([adj[1], pad_idx])
  row = row_flat.reshape(NC, NS, 2, cw // 2, CH)
  col = col_flat.reshape(NC, NS, 2, cw // 2, CH)
  x_pad = jnp.pad(x, ((0, n_pad - n), (0, 0)))
  e_pad_arr = jnp.pad(e, ((0, n_pad - n), (0, 0)))

  d_part = _make_hist(n_pad, cw)(col)

  rs, xp = pl.pallas_call(
      _prep_body,
      grid=(n_pad // blk,),
      in_specs=[
          pl.BlockSpec((NC, blk), lambda i: (0, i)),
          pl.BlockSpec((blk, f), lambda i: (i, 0)),
      ],
      out_specs=[
          pl.BlockSpec((blk, 1), lambda i: (i, 0)),
          pl.BlockSpec((blk, f), lambda i: (i, 0)),
      ],
      out_shape=[
          jax.ShapeDtypeStruct((n_pad, 1), jnp.float32),
          jax.ShapeDtypeStruct((n_pad, f), jnp.float32),
      ],
  )(d_part, x_pad)

  y_part = _make_scatter(n_pad, cw, f)(xp, row, col)

  xw = pl.pallas_call(
      functools.partial(_xw_body, f=f, k=k),
      grid=(n_pad // blk,),
      in_specs=[
          pl.BlockSpec((blk, f), lambda i: (i, 0)),
          pl.BlockSpec((blk, k), lambda i: (i, 0)),
          pl.BlockSpec((k, 2 * f, f), lambda i: (0, 0, 0)),
      ],
      out_specs=pl.BlockSpec((blk, f), lambda i: (i, 0)),
      out_shape=jax.ShapeDtypeStruct((n_pad, f), jnp.float32),
  )(x_pad, e_pad_arr, W)

  out = pl.pallas_call(
      functools.partial(_final_body, f=f, k=k),
      grid=(n_pad // blk,),
      in_specs=[
          pl.BlockSpec((blk, f), lambda i: (i, 0)),
          pl.BlockSpec((NC, blk, f), lambda i: (0, i, 0)),
          pl.BlockSpec((blk, 1), lambda i: (i, 0)),
          pl.BlockSpec((blk, k), lambda i: (i, 0)),
          pl.BlockSpec((k, 2 * f, f), lambda i: (0, 0, 0)),
      ],
      out_specs=pl.BlockSpec((blk, f), lambda i: (i, 0)),
      out_shape=jax.ShapeDtypeStruct((n, f), jnp.float32),
  )(xw, y_part, rs, e_pad_arr, W)

  return out


# speedup vs baseline: 1.2333x; 1.0267x over previous
import functools

import jax
import jax.numpy as jnp
from jax import lax
from jax.experimental import pallas as pl
from jax.experimental.pallas import tpu as pltpu
from jax.experimental.pallas import tpu_sc as plsc

NC = 2
NS = 16
NW = NC * NS
CH = 128


def _sc_mesh():
  return plsc.VectorSubcoreMesh(core_axis_name="c", subcore_axis_name="s")


def _make_hist(n_pad, cw):
  rows = n_pad // NS

  @functools.partial(
      pl.kernel,
      out_type=jax.ShapeDtypeStruct((NC, n_pad), jnp.float32),
      mesh=_sc_mesh(),
      scratch_types=[
          pltpu.VMEM((2, cw // 2, CH), jnp.int32),
          pltpu.VMEM((CH,), jnp.float32),
          pltpu.VMEM((rows,), jnp.float32),
          pltpu.SemaphoreType.DMA,
          pltpu.VMEM_SHARED((n_pad,), jnp.float32),
      ],
  )
  def hist(col_hbm, d_hbm, idx_v, ones_v, zer_v, sem, hist_sh):
    cid = lax.axis_index("c")
    sid = lax.axis_index("s")
    pltpu.sync_copy(col_hbm.at[cid, sid], idx_v)
    for i in range(CH // 16):
      ones_v[pl.ds(i * 16, 16)] = jnp.ones((16,), jnp.float32)

    def zfill(i, carry):
      zer_v[pl.ds(i * 16, 16)] = jnp.zeros((16,), jnp.float32)
      return carry

    lax.fori_loop(0, rows // 16, zfill, 0)
    pltpu.sync_copy(zer_v, hist_sh.at[pl.ds(sid * rows, rows)])
    plsc.subcore_barrier()

    for ph in range(2):
      def body(j, carry):
        pltpu.async_copy(ones_v, hist_sh.at[idx_v.at[ph, j]], sem, add=True)
        return carry

      lax.fori_loop(0, cw // 2, body, 0)

    for ph in range(2):
      def drain(j, carry):
        pltpu.make_async_copy(ones_v, hist_sh.at[idx_v.at[ph, j]], sem).wait()
        return carry

      lax.fori_loop(0, cw // 2, drain, 0)

    plsc.subcore_barrier()
    pltpu.sync_copy(hist_sh.at[pl.ds(sid * rows, rows)],
                    d_hbm.at[cid, pl.ds(sid * rows, rows)])

  return hist


def _make_scatter(n_pad, cw, f):
  rows = n_pad // NS

  @functools.partial(
      pl.kernel,
      out_type=jax.ShapeDtypeStruct((NC, n_pad, f), jnp.float32),
      mesh=_sc_mesh(),
      scratch_types=[
          pltpu.VMEM((cw // 2, CH), jnp.int32),
          pltpu.VMEM((cw // 2, CH), jnp.int32),
          pltpu.SemaphoreType.DMA,
          pltpu.SemaphoreType.DMA,
          pltpu.VMEM_SHARED((n_pad, f), jnp.float32),
      ],
  )
  def scatter(xp_hbm, row_hbm, col_hbm, y_hbm,
              rowv, colv, sem0, sem1, y_sh):
    cid = lax.axis_index("c")
    sid = lax.axis_index("s")

    def run(buf0, buf1):
      scatter_body(xp_hbm, row_hbm, col_hbm, y_hbm,
                   rowv, colv, buf0, buf1, sem0, sem1, y_sh,
                   cid, sid, cw, rows, f)

    pl.run_scoped(
        run,
        pltpu.VMEM((CH, f), jnp.float32),
        pltpu.VMEM((CH, f), jnp.float32),
    )

  return scatter


def scatter_body(xp_hbm, row_hbm, col_hbm, y_hbm,
                 rowv, colv, buf0, buf1, sem0, sem1, y_sh,
                 cid, sid, cw, rows, f):
    zr = 64

    def zfill(r, carry):
      for i in range(f // 16):
        buf0[r, pl.ds(i * 16, 16)] = jnp.zeros((16,), jnp.float32)
      return carry

    lax.fori_loop(0, zr, zfill, 0)

    def zcopy(r, carry):
      pltpu.sync_copy(buf0.at[pl.ds(0, zr)],
                      y_sh.at[pl.ds(sid * rows + r * zr, zr)])
      return carry

    lax.fori_loop(0, rows // zr, zcopy, 0)
    plsc.subcore_barrier()

    npair = cw // 4

    for ph in range(2):
      pltpu.sync_copy(row_hbm.at[cid, sid, ph], rowv)
      pltpu.sync_copy(col_hbm.at[cid, sid, ph], colv)
      pltpu.async_copy(xp_hbm.at[rowv.at[0]], buf0, sem0)
      pltpu.async_copy(xp_hbm.at[rowv.at[1]], buf1, sem1)

      def body(g, carry):
        j0 = 2 * g

        pltpu.make_async_copy(xp_hbm.at[rowv.at[j0]], buf0, sem0).wait()
        pltpu.sync_copy(buf0, y_sh.at[colv.at[j0]], add=True)

        @pl.when(g + 1 < npair)
        def _():
          pltpu.async_copy(xp_hbm.at[rowv.at[j0 + 2]], buf0, sem0)

        pltpu.make_async_copy(xp_hbm.at[rowv.at[j0 + 1]], buf1, sem1).wait()
        pltpu.sync_copy(buf1, y_sh.at[colv.at[j0 + 1]], add=True)

        @pl.when(g + 1 < npair)
        def _():
          pltpu.async_copy(xp_hbm.at[rowv.at[j0 + 3]], buf1, sem1)

        return carry

      lax.fori_loop(0, npair, body, 0)

    plsc.subcore_barrier()
    pltpu.sync_copy(y_sh.at[pl.ds(sid * rows, rows)],
                    y_hbm.at[cid, pl.ds(sid * rows, rows)])


def _prep_body(d_ref, x_ref, rs_ref, xp_ref):
  d = d_ref[0, :] + d_ref[1, :]
  rs = jnp.where(d > 0.0, lax.rsqrt(d), 0.0)
  rs2 = rs[:, None]
  rs_ref[...] = rs2
  xp_ref[...] = rs2 * x_ref[...]


def _xw_body(x_ref, e_ref, w_ref, o_ref, *, f, k):
  x = x_ref[...]
  w = w_ref[...]
  e = e_ref[...]
  acc = x
  for i in range(k):
    hk = jnp.dot(x, w[i, f:, :], preferred_element_type=jnp.float32)
    acc = acc + e[:, i:i + 1] * hk
  o_ref[...] = acc


def _final_body(xw_ref, yp_ref, rs_ref, e_ref, w_ref, o_ref, *, f, k):
  y = rs_ref[...] * (yp_ref[0] + yp_ref[1])
  w = w_ref[...]
  e = e_ref[...]
  acc = xw_ref[...]
  for i in range(k):
    hk = jnp.dot(y, w[i, :f, :], preferred_element_type=jnp.float32)
    acc = acc + e[:, i:i + 1] * hk
  o_ref[...] = acc


@jax.jit
def kernel(x, adj, e, W):
  n, f = x.shape
  k = W.shape[0]
  eN = adj.shape[1]

  n_pad = ((n + 1024) // 1024) * 1024
  blk = 2048 if n_pad % 2048 == 0 else 1024
  e_chunk = NW * CH
  cw = (eN + e_chunk - 1) // e_chunk
  cw = ((cw + 3) // 4) * 4
  e_pad = cw * e_chunk

  pad_idx = (n + jnp.arange(e_pad - eN, dtype=jnp.int32) % (n_pad - n)).astype(jnp.int32)
  row_flat = jnp.concatenate([adj[0], pad_idx])
  col_flat = jnp.concatenate([adj[1], pad_idx])
  row = row_flat.reshape(NC, NS, 2, cw // 2, CH)
  col = col_flat.reshape(NC, NS, 2, cw // 2, CH)
  d_part = _make_hist(n_pad, cw)(col)

  rs, xp = pl.pallas_call(
      _prep_body,
      grid=(n_pad // blk,),
      in_specs=[
          pl.BlockSpec((NC, blk), lambda i: (0, i)),
          pl.BlockSpec((blk, f), lambda i: (i, 0)),
      ],
      out_specs=[
          pl.BlockSpec((blk, 1), lambda i: (i, 0)),
          pl.BlockSpec((blk, f), lambda i: (i, 0)),
      ],
      out_shape=[
          jax.ShapeDtypeStruct((n_pad, 1), jnp.float32),
          jax.ShapeDtypeStruct((n_pad, f), jnp.float32),
      ],
  )(d_part, x)

  y_part = _make_scatter(n_pad, cw, f)(xp, row, col)

  xw = pl.pallas_call(
      functools.partial(_xw_body, f=f, k=k),
      grid=(n_pad // blk,),
      in_specs=[
          pl.BlockSpec((blk, f), lambda i: (i, 0)),
          pl.BlockSpec((blk, k), lambda i: (i, 0)),
          pl.BlockSpec((k, 2 * f, f), lambda i: (0, 0, 0)),
      ],
      out_specs=pl.BlockSpec((blk, f), lambda i: (i, 0)),
      out_shape=jax.ShapeDtypeStruct((n_pad, f), jnp.float32),
  )(x, e, W)

  out = pl.pallas_call(
      functools.partial(_final_body, f=f, k=k),
      grid=(n_pad // blk,),
      in_specs=[
          pl.BlockSpec((blk, f), lambda i: (i, 0)),
          pl.BlockSpec((NC, blk, f), lambda i: (0, i, 0)),
          pl.BlockSpec((blk, 1), lambda i: (i, 0)),
          pl.BlockSpec((blk, k), lambda i: (i, 0)),
          pl.BlockSpec((k, 2 * f, f), lambda i: (0, 0, 0)),
      ],
      out_specs=pl.BlockSpec((blk, f), lambda i: (i, 0)),
      out_shape=jax.ShapeDtypeStruct((n, f), jnp.float32),
  )(xw, y_part, rs, e, W)

  return out


# speedup vs baseline: 1.2393x; 1.0048x over previous
import functools

import jax
import jax.numpy as jnp
import numpy as np
from jax import lax
from jax.experimental import pallas as pl
from jax.experimental.pallas import tpu as pltpu
from jax.experimental.pallas import tpu_sc as plsc

NC = 2
NS = 16
NW = NC * NS
CH = 128


def _sc_mesh():
  return plsc.VectorSubcoreMesh(core_axis_name="c", subcore_axis_name="s")


def _make_hist(n_pad, cw):
  rows = n_pad // NS

  @functools.partial(
      pl.kernel,
      out_type=jax.ShapeDtypeStruct((NC, n_pad), jnp.float32),
      mesh=_sc_mesh(),
      scratch_types=[
          pltpu.VMEM((2, cw // 2, CH), jnp.int32),
          pltpu.VMEM((CH,), jnp.float32),
          pltpu.VMEM((rows,), jnp.float32),
          pltpu.SemaphoreType.DMA,
          pltpu.VMEM_SHARED((n_pad,), jnp.float32),
      ],
  )
  def hist(col_hbm, d_hbm, idx_v, ones_v, zer_v, sem, hist_sh):
    cid = lax.axis_index("c")
    sid = lax.axis_index("s")
    pltpu.sync_copy(col_hbm.at[cid, sid], idx_v)
    for i in range(CH // 16):
      ones_v[pl.ds(i * 16, 16)] = jnp.ones((16,), jnp.float32)

    def zfill(i, carry):
      zer_v[pl.ds(i * 16, 16)] = jnp.zeros((16,), jnp.float32)
      return carry

    lax.fori_loop(0, rows // 16, zfill, 0)
    pltpu.sync_copy(zer_v, hist_sh.at[pl.ds(sid * rows, rows)])
    plsc.subcore_barrier()

    for ph in range(2):
      def body(j, carry):
        pltpu.async_copy(ones_v, hist_sh.at[idx_v.at[ph, j]], sem, add=True)
        return carry

      lax.fori_loop(0, cw // 2, body, 0)

    for ph in range(2):
      def drain(j, carry):
        pltpu.make_async_copy(ones_v, hist_sh.at[idx_v.at[ph, j]], sem).wait()
        return carry

      lax.fori_loop(0, cw // 2, drain, 0)

    plsc.subcore_barrier()
    pltpu.sync_copy(hist_sh.at[pl.ds(sid * rows, rows)],
                    d_hbm.at[cid, pl.ds(sid * rows, rows)])

  return hist


def _make_scatter(n_pad, cw, f):
  rows = n_pad // NS

  @functools.partial(
      pl.kernel,
      out_type=jax.ShapeDtypeStruct((NC, n_pad, f), jnp.float32),
      mesh=_sc_mesh(),
      scratch_types=[
          pltpu.VMEM((cw // 2, CH), jnp.int32),
          pltpu.VMEM((cw // 2, CH), jnp.int32),
          pltpu.SemaphoreType.DMA,
          pltpu.SemaphoreType.DMA,
          pltpu.VMEM_SHARED((n_pad, f), jnp.float32),
      ],
  )
  def scatter(xp_hbm, row_hbm, col_hbm, y_hbm,
              rowv, colv, sem0, sem1, y_sh):
    cid = lax.axis_index("c")
    sid = lax.axis_index("s")

    def run(buf0, buf1):
      scatter_body(xp_hbm, row_hbm, col_hbm, y_hbm,
                   rowv, colv, buf0, buf1, sem0, sem1, y_sh,
                   cid, sid, cw, rows, f)

    pl.run_scoped(
        run,
        pltpu.VMEM((CH, f), jnp.float32),
        pltpu.VMEM((CH, f), jnp.float32),
    )

  return scatter


def scatter_body(xp_hbm, row_hbm, col_hbm, y_hbm,
                 rowv, colv, buf0, buf1, sem0, sem1, y_sh,
                 cid, sid, cw, rows, f):
    zr = 64

    def zfill(r, carry):
      for i in range(f // 16):
        buf0[r, pl.ds(i * 16, 16)] = jnp.zeros((16,), jnp.float32)
      return carry

    lax.fori_loop(0, zr, zfill, 0)

    def zcopy(r, carry):
      pltpu.sync_copy(buf0.at[pl.ds(0, zr)],
                      y_sh.at[pl.ds(sid * rows + r * zr, zr)])
      return carry

    lax.fori_loop(0, rows // zr, zcopy, 0)
    plsc.subcore_barrier()

    npair = cw // 4

    for ph in range(2):
      pltpu.sync_copy(row_hbm.at[cid, sid, ph], rowv)
      pltpu.sync_copy(col_hbm.at[cid, sid, ph], colv)
      pltpu.async_copy(xp_hbm.at[rowv.at[0]], buf0, sem0)
      pltpu.async_copy(xp_hbm.at[rowv.at[1]], buf1, sem1)

      def body(g, carry):
        j0 = 2 * g

        pltpu.make_async_copy(xp_hbm.at[rowv.at[j0]], buf0, sem0).wait()
        pltpu.sync_copy(buf0, y_sh.at[colv.at[j0]], add=True)

        @pl.when(g + 1 < npair)
        def _():
          pltpu.async_copy(xp_hbm.at[rowv.at[j0 + 2]], buf0, sem0)

        pltpu.make_async_copy(xp_hbm.at[rowv.at[j0 + 1]], buf1, sem1).wait()
        pltpu.sync_copy(buf1, y_sh.at[colv.at[j0 + 1]], add=True)

        @pl.when(g + 1 < npair)
        def _():
          pltpu.async_copy(xp_hbm.at[rowv.at[j0 + 3]], buf1, sem1)

        return carry

      lax.fori_loop(0, npair, body, 0)

    plsc.subcore_barrier()
    pltpu.sync_copy(y_sh.at[pl.ds(sid * rows, rows)],
                    y_hbm.at[cid, pl.ds(sid * rows, rows)])


def _prep_body(d_ref, x_ref, rs_ref, xp_ref):
  d = d_ref[0, :] + d_ref[1, :]
  rs = jnp.where(d > 0.0, lax.rsqrt(d), 0.0)
  rs2 = rs[:, None]
  rs_ref[...] = rs2
  xp_ref[...] = rs2 * x_ref[...]


def _xw_body(x_ref, e_ref, w_ref, o_ref, *, f, k):
  x = x_ref[...]
  w = w_ref[...]
  e = e_ref[...]
  acc = x
  for i in range(k):
    hk = jnp.dot(x, w[i, f:, :], preferred_element_type=jnp.float32)
    acc = acc + e[:, i:i + 1] * hk
  o_ref[...] = acc


def _final_body(xw_ref, yp_ref, rs_ref, e_ref, w_ref, o_ref, *, f, k):
  y = rs_ref[...] * (yp_ref[0] + yp_ref[1])
  w = w_ref[...]
  e = e_ref[...]
  acc = xw_ref[...]
  for i in range(k):
    hk = jnp.dot(y, w[i, :f, :], preferred_element_type=jnp.float32)
    acc = acc + e[:, i:i + 1] * hk
  o_ref[...] = acc


@jax.jit
def kernel(x, adj, e, W):
  n, f = x.shape
  k = W.shape[0]
  eN = adj.shape[1]

  n_pad = ((n + 1024) // 1024) * 1024
  blk = 2048 if n_pad % 2048 == 0 else 1024
  e_chunk = NW * CH
  cw = (eN + e_chunk - 1) // e_chunk
  cw = ((cw + 3) // 4) * 4
  e_pad = cw * e_chunk

  pad_idx = jnp.asarray(n + np.arange(e_pad - eN) % (n_pad - n), dtype=jnp.int32)
  row_flat = jnp.concatenate([adj[0], pad_idx])
  col_flat = jnp.concatenate([adj[1], pad_idx])
  row = row_flat.reshape(NC, NS, 2, cw // 2, CH)
  col = col_flat.reshape(NC, NS, 2, cw // 2, CH)
  d_part = _make_hist(n_pad, cw)(col)

  rs, xp = pl.pallas_call(
      _prep_body,
      grid=(n_pad // blk,),
      in_specs=[
          pl.BlockSpec((NC, blk), lambda i: (0, i)),
          pl.BlockSpec((blk, f), lambda i: (i, 0)),
      ],
      out_specs=[
          pl.BlockSpec((blk, 1), lambda i: (i, 0)),
          pl.BlockSpec((blk, f), lambda i: (i, 0)),
      ],
      out_shape=[
          jax.ShapeDtypeStruct((n_pad, 1), jnp.float32),
          jax.ShapeDtypeStruct((n_pad, f), jnp.float32),
      ],
  )(d_part, x)

  y_part = _make_scatter(n_pad, cw, f)(xp, row, col)

  xw = pl.pallas_call(
      functools.partial(_xw_body, f=f, k=k),
      grid=(n_pad // blk,),
      in_specs=[
          pl.BlockSpec((blk, f), lambda i: (i, 0)),
          pl.BlockSpec((blk, k), lambda i: (i, 0)),
          pl.BlockSpec((k, 2 * f, f), lambda i: (0, 0, 0)),
      ],
      out_specs=pl.BlockSpec((blk, f), lambda i: (i, 0)),
      out_shape=jax.ShapeDtypeStruct((n_pad, f), jnp.float32),
  )(x, e, W)

  out = pl.pallas_call(
      functools.partial(_final_body, f=f, k=k),
      grid=(n_pad // blk,),
      in_specs=[
          pl.BlockSpec((blk, f), lambda i: (i, 0)),
          pl.BlockSpec((NC, blk, f), lambda i: (0, i, 0)),
          pl.BlockSpec((blk, 1), lambda i: (i, 0)),
          pl.BlockSpec((blk, k), lambda i: (i, 0)),
          pl.BlockSpec((k, 2 * f, f), lambda i: (0, 0, 0)),
      ],
      out_specs=pl.BlockSpec((blk, f), lambda i: (i, 0)),
      out_shape=jax.ShapeDtypeStruct((n, f), jnp.float32),
  )(xw, y_part, rs, e, W)

  return out
